# baseline (speedup 1.0000x reference)
import functools

import jax
import jax.numpy as jnp
from jax import lax
from jax.experimental import pallas as pl
from jax.experimental.pallas import tpu as pltpu
from jax.experimental.pallas import tpu_sc as plsc

N = 10000
NPAD = 10240
E = 320000
D_IN = 128
D_HID = 128
N_CLS = 64

CHUNK = 128
SPT = 160
NSTAGE = 4
SCH = SPT // NSTAGE
NBUF = 4
LAG = 2
EPAD = 16 * SPT * CHUNK
RPS = NPAD // 16

_mesh = plsc.VectorSubcoreMesh(core_axis_name="c", subcore_axis_name="s")
_sc_params = pltpu.CompilerParams(use_tc_tiling_on_sc=False)


def _make_agg(D):
  D2 = D // 2

  @functools.partial(
      pl.kernel,
      out_type=jax.ShapeDtypeStruct((NPAD, D), jnp.float32),
      mesh=_mesh,
      scratch_types=[
          pltpu.VMEM((SCH, CHUNK), jnp.int32),
          pltpu.VMEM((SCH, CHUNK), jnp.int32),
          [pltpu.VMEM((CHUNK, D2), jnp.float32) for _ in range(NBUF)],
          pltpu.VMEM_SHARED((NPAD, D2), jnp.float32),
          pltpu.VMEM_SHARED((NPAD, D2), jnp.float32),
          [pltpu.SemaphoreType.DMA for _ in range(NBUF)],
          [pltpu.SemaphoreType.DMA for _ in range(NBUF)],
      ],
      compiler_params=_sc_params,
  )
  def agg(g_hbm, src_hbm, dst_hbm, zeros_hbm, out_hbm,
          src_v, dst_v, bufs, op_sh, acc_sh, gsems, ssems):
    c = lax.axis_index("c")
    s = lax.axis_index("s")
    pltpu.sync_copy(g_hbm.at[pl.ds(s * RPS, RPS), pl.ds(c * D2, D2)],
                    op_sh.at[pl.ds(s * RPS, RPS)])
    pltpu.sync_copy(zeros_hbm, acc_sh.at[pl.ds(s * RPS, RPS)])
    plsc.subcore_barrier()

    def body(st, carry):
      for j in range(NBUF):
        ch = st * NBUF + j
        b = j
        pltpu.make_async_copy(op_sh.at[src_v.at[ch]], bufs[b], gsems[b]).wait()
        pltpu.async_copy(bufs[b], acc_sh.at[dst_v.at[ch]], ssems[b], add=True)
        b2 = (j + LAG) % NBUF
        nxt = ch + LAG

        @pl.when(jnp.logical_and(nxt >= NBUF, nxt < SCH))
        def _():
          pltpu.make_async_copy(
              bufs[b2], acc_sh.at[dst_v.at[ch]], ssems[b2]).wait()

        @pl.when(nxt < SCH)
        def _():
          pltpu.async_copy(op_sh.at[src_v.at[nxt]], bufs[b2], gsems[b2])

      return carry

    for stage in range(NSTAGE):
      pltpu.sync_copy(src_hbm.at[s, stage], src_v)
      pltpu.sync_copy(dst_hbm.at[s, stage], dst_v)
      for b in range(LAG):
        pltpu.async_copy(op_sh.at[src_v.at[b]], bufs[b], gsems[b])
      lax.fori_loop(0, SCH // NBUF, body, 0)
      for b in range(NBUF):
        pltpu.make_async_copy(bufs[b], acc_sh.at[dst_v.at[b]], ssems[b]).wait()
    plsc.subcore_barrier()
    pltpu.sync_copy(acc_sh.at[pl.ds(s * RPS, RPS)],
                    out_hbm.at[pl.ds(s * RPS, RPS), pl.ds(c * D2, D2)])

  return agg


_agg128 = _make_agg(D_HID)
_agg64 = _make_agg(N_CLS)


@functools.partial(
    pl.kernel,
    out_type=jax.ShapeDtypeStruct((NPAD, 128), jnp.float32),
    mesh=_mesh,
    scratch_types=[
        pltpu.VMEM((SCH, CHUNK), jnp.int32),
        pltpu.VMEM((CHUNK, 64), jnp.float32),
        pltpu.VMEM_SHARED((NPAD, 64), jnp.float32),
    ],
    compiler_params=_sc_params,
)
def _deg_kernel(dst_hbm, onehot_hbm, zeros_hbm, out_hbm, dst_v, ones_v, acc_sh):
  c = lax.axis_index("c")
  s = lax.axis_index("s")
  pltpu.sync_copy(zeros_hbm, acc_sh.at[pl.ds(s * RPS, RPS)])
  pltpu.sync_copy(onehot_hbm, ones_v)
  plsc.subcore_barrier()

  def body(ch, carry):
    pltpu.sync_copy(ones_v, acc_sh.at[dst_v.at[ch]], add=True)
    return carry

  for k in range(NSTAGE // 2):
    pltpu.sync_copy(dst_hbm.at[s, c + 2 * k], dst_v)
    lax.fori_loop(0, SCH, body, 0)
  plsc.subcore_barrier()
  pltpu.sync_copy(acc_sh.at[pl.ds(s * RPS, RPS)],
                  out_hbm.at[pl.ds(s * RPS, RPS), pl.ds(c * 64, 64)])


def _isd_of(deg_ref):
  deg = deg_ref[:, 0:1] + deg_ref[:, 64:65]
  return lax.rsqrt(jnp.maximum(deg, 1.0))


def _mm1_body(x_ref, w_ref, deg_ref, o_ref):
  o_ref[...] = jnp.dot(x_ref[...], w_ref[...],
                       preferred_element_type=jnp.float32) * _isd_of(deg_ref)


def _mm2_body(a_ref, deg_ref, b1_ref, w_ref, o_ref):
  isd = _isd_of(deg_ref)
  h = jnp.maximum(isd * a_ref[...] + b1_ref[...], 0.0)
  o_ref[...] = jnp.dot(h, w_ref[...],
                       preferred_element_type=jnp.float32) * isd


_BLK = 512
_GRID = NPAD // _BLK


def _row_spec(d):
  return pl.BlockSpec((_BLK, d), lambda i: (i, 0))


def _full_spec(r, c):
  return pl.BlockSpec((r, c), lambda i: (0, 0))


_mm1 = pl.pallas_call(
    _mm1_body,
    grid=(_GRID,),
    in_specs=[_row_spec(D_IN), _full_spec(D_IN, D_HID), _row_spec(128)],
    out_specs=_row_spec(D_HID),
    out_shape=jax.ShapeDtypeStruct((NPAD, D_HID), jnp.float32),
)

_mm2 = pl.pallas_call(
    _mm2_body,
    grid=(_GRID,),
    in_specs=[_row_spec(D_HID), _row_spec(128),
              _full_spec(1, D_HID), _full_spec(D_HID, N_CLS)],
    out_specs=_row_spec(N_CLS),
    out_shape=jax.ShapeDtypeStruct((NPAD, N_CLS), jnp.float32),
)


def kernel(x, edge_index, W1, b1, W2, b2):
  src = edge_index[0].astype(jnp.int32)
  dst = edge_index[1].astype(jnp.int32)
  src4 = jnp.concatenate([src, jnp.zeros((EPAD - E,), jnp.int32)])
  dst4 = jnp.concatenate([dst, jnp.full((EPAD - E,), NPAD - 1, jnp.int32)])
  src4 = src4.reshape(16, NSTAGE, SCH, CHUNK)
  dst4 = dst4.reshape(16, NSTAGE, SCH, CHUNK)
  xp = jnp.zeros((NPAD, D_IN), jnp.float32).at[:N].set(x)

  onehot = jnp.zeros((CHUNK, 64), jnp.float32).at[:, 0].set(1.0)
  z64 = jnp.zeros((RPS, D_HID // 2), jnp.float32)
  z32 = jnp.zeros((RPS, N_CLS // 2), jnp.float32)

  deg = _deg_kernel(dst4, onehot, z64)
  g1 = _mm1(xp, W1, deg)
  agg1 = _agg128(g1, src4, dst4, z64)
  g2 = _mm2(agg1, deg, b1.reshape(1, D_HID), W2)
  agg2 = _agg64(g2, src4, dst4, z32)

  isd = lax.rsqrt(jnp.maximum(deg[:, 0:1] + deg[:, 64:65], 1.0))
  return (isd * agg2 + b2)[:N]


# speedup vs baseline: 1.0107x; 1.0107x over previous
import functools

import jax
import jax.numpy as jnp
from jax import lax
from jax.experimental import pallas as pl
from jax.experimental.pallas import tpu as pltpu
from jax.experimental.pallas import tpu_sc as plsc

N = 10000
NPAD = 10240
E = 320000
D_IN = 128
D_HID = 128
N_CLS = 64

CHUNK = 128
SPT = 160
NSTAGE = 4
SCH = SPT // NSTAGE
NBUF = 4
LAG = 2
EPAD = 16 * SPT * CHUNK
RPS = NPAD // 16

_mesh = plsc.VectorSubcoreMesh(core_axis_name="c", subcore_axis_name="s")
_sc_params = pltpu.CompilerParams(use_tc_tiling_on_sc=False)


def _make_agg(D):
  D2 = D // 2

  @functools.partial(
      pl.kernel,
      out_type=jax.ShapeDtypeStruct((NPAD, D), jnp.float32),
      mesh=_mesh,
      scratch_types=[
          pltpu.VMEM((SCH, CHUNK), jnp.int32),
          pltpu.VMEM((SCH, CHUNK), jnp.int32),
          [pltpu.VMEM((CHUNK, D2), jnp.float32) for _ in range(NBUF)],
          pltpu.VMEM_SHARED((NPAD, D2), jnp.float32),
          pltpu.VMEM_SHARED((NPAD, D2), jnp.float32),
          [pltpu.SemaphoreType.DMA for _ in range(NBUF)],
          [pltpu.SemaphoreType.DMA for _ in range(NBUF)],
      ],
      compiler_params=_sc_params,
  )
  def agg(g_hbm, src_hbm, dst_hbm, zeros_hbm, out_hbm,
          src_v, dst_v, bufs, op_sh, acc_sh, gsems, ssems):
    c = lax.axis_index("c")
    s = lax.axis_index("s")
    pltpu.sync_copy(g_hbm.at[pl.ds(s * RPS, RPS), pl.ds(c * D2, D2)],
                    op_sh.at[pl.ds(s * RPS, RPS)])
    pltpu.sync_copy(zeros_hbm, acc_sh.at[pl.ds(s * RPS, RPS)])
    plsc.subcore_barrier()

    def body(st, carry):
      for j in range(NBUF):
        ch = st * NBUF + j
        b = j
        pltpu.make_async_copy(op_sh.at[src_v.at[ch]], bufs[b], gsems[b]).wait()
        pltpu.async_copy(bufs[b], acc_sh.at[dst_v.at[ch]], ssems[b], add=True)
        b2 = (j + LAG) % NBUF
        nxt = ch + LAG

        @pl.when(jnp.logical_and(nxt >= NBUF, nxt < SCH))
        def _():
          pltpu.make_async_copy(
              bufs[b2], acc_sh.at[dst_v.at[ch]], ssems[b2]).wait()

        @pl.when(nxt < SCH)
        def _():
          pltpu.async_copy(op_sh.at[src_v.at[nxt]], bufs[b2], gsems[b2])

      return carry

    for stage in range(NSTAGE):
      pltpu.sync_copy(src_hbm.at[s, stage], src_v)
      pltpu.sync_copy(dst_hbm.at[s, stage], dst_v)
      for b in range(LAG):
        pltpu.async_copy(op_sh.at[src_v.at[b]], bufs[b], gsems[b])
      lax.fori_loop(0, SCH // NBUF, body, 0)
      for b in range(NBUF):
        pltpu.make_async_copy(bufs[b], acc_sh.at[dst_v.at[b]], ssems[b]).wait()
    plsc.subcore_barrier()
    pltpu.sync_copy(acc_sh.at[pl.ds(s * RPS, RPS)],
                    out_hbm.at[pl.ds(s * RPS, RPS), pl.ds(c * D2, D2)])

  return agg


_agg128 = _make_agg(D_HID)
_agg64 = _make_agg(N_CLS)


@functools.partial(
    pl.kernel,
    out_type=jax.ShapeDtypeStruct((NPAD, 128), jnp.float32),
    mesh=_mesh,
    scratch_types=[
        pltpu.VMEM((SCH, CHUNK), jnp.int32),
        pltpu.VMEM((CHUNK, 64), jnp.float32),
        pltpu.VMEM_SHARED((NPAD, 64), jnp.float32),
    ],
    compiler_params=_sc_params,
)
def _deg_kernel(dst_hbm, onehot_hbm, zeros_hbm, out_hbm, dst_v, ones_v, acc_sh):
  c = lax.axis_index("c")
  s = lax.axis_index("s")
  pltpu.sync_copy(zeros_hbm, acc_sh.at[pl.ds(s * RPS, RPS)])
  pltpu.sync_copy(onehot_hbm, ones_v)
  plsc.subcore_barrier()

  def body(ch, carry):
    pltpu.sync_copy(ones_v, acc_sh.at[dst_v.at[ch]], add=True)
    return carry

  for k in range(NSTAGE // 2):
    pltpu.sync_copy(dst_hbm.at[s, c + 2 * k], dst_v)
    lax.fori_loop(0, SCH, body, 0)
  plsc.subcore_barrier()
  pltpu.sync_copy(acc_sh.at[pl.ds(s * RPS, RPS)],
                  out_hbm.at[pl.ds(s * RPS, RPS), pl.ds(c * 64, 64)])


def _isd_of(deg_ref):
  deg = deg_ref[:, 0:1] + deg_ref[:, 64:65]
  return lax.rsqrt(jnp.maximum(deg, 1.0))


def _mm1_body(x_ref, w_ref, o_ref):
  o_ref[...] = jnp.dot(x_ref[...], w_ref[...],
                       preferred_element_type=jnp.float32)


def _mm2_body(a_ref, deg_ref, b1_ref, w_ref, o_ref):
  isd = _isd_of(deg_ref)
  h = jnp.maximum(isd * a_ref[...] + b1_ref[...], 0.0)
  o_ref[...] = jnp.dot(h, w_ref[...],
                       preferred_element_type=jnp.float32) * isd


_BLK = 512
_GRID = NPAD // _BLK


def _row_spec(d):
  return pl.BlockSpec((_BLK, d), lambda i: (i, 0))


def _full_spec(r, c):
  return pl.BlockSpec((r, c), lambda i: (0, 0))


_mm1 = pl.pallas_call(
    _mm1_body,
    grid=(_GRID,),
    in_specs=[_row_spec(D_IN), _full_spec(D_IN, D_HID)],
    out_specs=_row_spec(D_HID),
    out_shape=jax.ShapeDtypeStruct((NPAD, D_HID), jnp.float32),
)

_mm2 = pl.pallas_call(
    _mm2_body,
    grid=(_GRID,),
    in_specs=[_row_spec(D_HID), _row_spec(128),
              _full_spec(1, D_HID), _full_spec(D_HID, N_CLS)],
    out_specs=_row_spec(N_CLS),
    out_shape=jax.ShapeDtypeStruct((NPAD, N_CLS), jnp.float32),
)


def kernel(x, edge_index, W1, b1, W2, b2):
  src = edge_index[0].astype(jnp.int32)
  dst = edge_index[1].astype(jnp.int32)
  src4 = jnp.concatenate([src, jnp.zeros((EPAD - E,), jnp.int32)])
  dst4 = jnp.concatenate([dst, jnp.full((EPAD - E,), NPAD - 1, jnp.int32)])
  src4 = src4.reshape(16, NSTAGE, SCH, CHUNK)
  dst4 = dst4.reshape(16, NSTAGE, SCH, CHUNK)
  xp = jnp.zeros((NPAD, D_IN), jnp.float32).at[:N].set(x)

  onehot = jnp.zeros((CHUNK, 64), jnp.float32).at[:, 0].set(1.0)
  z64 = jnp.zeros((RPS, D_HID // 2), jnp.float32)
  z32 = jnp.zeros((RPS, N_CLS // 2), jnp.float32)

  deg = _deg_kernel(dst4, onehot, z64)
  raw1 = _mm1(xp, W1)

  isd = lax.rsqrt(jnp.maximum(deg[:, 0:1] + deg[:, 64:65], 1.0))
  g1 = raw1 * isd
  agg1 = _agg128(g1, src4, dst4, z64)
  g2 = _mm2(agg1, deg, b1.reshape(1, D_HID), W2)
  agg2 = _agg64(g2, src4, dst4, z32)
  return (isd * agg2 + b2)[:N]


# speedup vs baseline: 1.0462x; 1.0351x over previous
import functools

import jax
import jax.numpy as jnp
from jax import lax
from jax.experimental import pallas as pl
from jax.experimental.pallas import tpu as pltpu
from jax.experimental.pallas import tpu_sc as plsc

N = 10000
NPAD = 10240
E = 320000
D_IN = 128
D_HID = 128
N_CLS = 64

CHUNK = 128
SPT = 160
NSTAGE = 4
SCH = SPT // NSTAGE
NBUF = 4
LAG = 2
EPAD = 16 * SPT * CHUNK
RPS = NPAD // 16

_mesh = plsc.VectorSubcoreMesh(core_axis_name="c", subcore_axis_name="s")
_sc_params = pltpu.CompilerParams(use_tc_tiling_on_sc=False)


def _make_agg(D):
  D2 = D // 2

  @functools.partial(
      pl.kernel,
      out_type=jax.ShapeDtypeStruct((NPAD, D), jnp.float32),
      mesh=_mesh,
      scratch_types=[
          pltpu.VMEM((SCH, CHUNK), jnp.int32),
          pltpu.VMEM((SCH, CHUNK), jnp.int32),
          [pltpu.VMEM((CHUNK, D2), jnp.float32) for _ in range(NBUF)],
          pltpu.VMEM_SHARED((NPAD, D2), jnp.float32),
          pltpu.VMEM_SHARED((NPAD, D2), jnp.float32),
          [pltpu.SemaphoreType.DMA for _ in range(NBUF)],
          [pltpu.SemaphoreType.DMA for _ in range(NBUF)],
      ],
      compiler_params=_sc_params,
  )
  def agg(g_hbm, src_hbm, dst_hbm, zeros_hbm, out_hbm,
          src_v, dst_v, bufs, op_sh, acc_sh, gsems, ssems):
    c = lax.axis_index("c")
    s = lax.axis_index("s")
    pltpu.sync_copy(g_hbm.at[pl.ds(s * RPS, RPS), pl.ds(c * D2, D2)],
                    op_sh.at[pl.ds(s * RPS, RPS)])
    pltpu.sync_copy(zeros_hbm, acc_sh.at[pl.ds(s * RPS, RPS)])
    plsc.subcore_barrier()

    def body(st, carry):
      for j in range(NBUF):
        ch = st * NBUF + j
        b = j
        pltpu.make_async_copy(op_sh.at[src_v.at[ch]], bufs[b], gsems[b]).wait()
        pltpu.async_copy(bufs[b], acc_sh.at[dst_v.at[ch]], ssems[b], add=True)
        b2 = (j + LAG) % NBUF
        nxt = ch + LAG

        @pl.when(jnp.logical_and(nxt >= NBUF, nxt < SCH))
        def _():
          pltpu.make_async_copy(
              bufs[b2], acc_sh.at[dst_v.at[ch]], ssems[b2]).wait()

        @pl.when(nxt < SCH)
        def _():
          pltpu.async_copy(op_sh.at[src_v.at[nxt]], bufs[b2], gsems[b2])

      return carry

    for stage in range(NSTAGE):
      pltpu.sync_copy(src_hbm.at[s, stage], src_v)
      pltpu.sync_copy(dst_hbm.at[s, stage], dst_v)
      for b in range(LAG):
        pltpu.async_copy(op_sh.at[src_v.at[b]], bufs[b], gsems[b])
      lax.fori_loop(0, SCH // NBUF, body, 0)
      for b in range(NBUF):
        pltpu.make_async_copy(bufs[b], acc_sh.at[dst_v.at[b]], ssems[b]).wait()
    plsc.subcore_barrier()
    pltpu.sync_copy(acc_sh.at[pl.ds(s * RPS, RPS)],
                    out_hbm.at[pl.ds(s * RPS, RPS), pl.ds(c * D2, D2)])

  return agg


_agg128 = _make_agg(D_HID)
_agg64 = _make_agg(N_CLS)


@functools.partial(
    pl.kernel,
    out_type=jax.ShapeDtypeStruct((2, NPAD, 16), jnp.float32),
    mesh=_mesh,
    scratch_types=[
        pltpu.VMEM((SCH, CHUNK), jnp.int32),
        pltpu.VMEM((CHUNK, 16), jnp.float32),
        pltpu.VMEM_SHARED((NPAD, 16), jnp.float32),
    ],
    compiler_params=_sc_params,
)
def _deg_kernel(dst_hbm, onehot_hbm, zeros_hbm, out_hbm, dst_v, ones_v, acc_sh):
  c = lax.axis_index("c")
  s = lax.axis_index("s")
  pltpu.sync_copy(zeros_hbm, acc_sh.at[pl.ds(s * RPS, RPS)])
  pltpu.sync_copy(onehot_hbm, ones_v)
  plsc.subcore_barrier()

  def body(ch, carry):
    pltpu.sync_copy(ones_v, acc_sh.at[dst_v.at[ch]], add=True)
    return carry

  for k in range(NSTAGE // 2):
    pltpu.sync_copy(dst_hbm.at[s, c + 2 * k], dst_v)
    lax.fori_loop(0, SCH, body, 0)
  plsc.subcore_barrier()
  pltpu.sync_copy(acc_sh.at[pl.ds(s * RPS, RPS)],
                  out_hbm.at[c, pl.ds(s * RPS, RPS)])


def _mm1_body(x_ref, w_ref, o_ref):
  o_ref[...] = jnp.dot(x_ref[...], w_ref[...],
                       preferred_element_type=jnp.float32)


def _mm2_body(h_ref, w_ref, o_ref):
  o_ref[...] = jnp.dot(h_ref[...], w_ref[...],
                       preferred_element_type=jnp.float32)


_BLK = 512
_GRID = NPAD // _BLK


def _row_spec(d):
  return pl.BlockSpec((_BLK, d), lambda i: (i, 0))


def _full_spec(r, c):
  return pl.BlockSpec((r, c), lambda i: (0, 0))


_mm1 = pl.pallas_call(
    _mm1_body,
    grid=(_GRID,),
    in_specs=[_row_spec(D_IN), _full_spec(D_IN, D_HID)],
    out_specs=_row_spec(D_HID),
    out_shape=jax.ShapeDtypeStruct((NPAD, D_HID), jnp.float32),
)

_mm2 = pl.pallas_call(
    _mm2_body,
    grid=(_GRID,),
    in_specs=[_row_spec(D_HID), _full_spec(D_HID, N_CLS)],
    out_specs=_row_spec(N_CLS),
    out_shape=jax.ShapeDtypeStruct((NPAD, N_CLS), jnp.float32),
)


def kernel(x, edge_index, W1, b1, W2, b2):
  src = edge_index[0].astype(jnp.int32)
  dst = edge_index[1].astype(jnp.int32)
  src4 = jnp.concatenate([src, jnp.zeros((EPAD - E,), jnp.int32)])
  dst4 = jnp.concatenate([dst, jnp.full((EPAD - E,), NPAD - 1, jnp.int32)])
  src4 = src4.reshape(16, NSTAGE, SCH, CHUNK)
  dst4 = dst4.reshape(16, NSTAGE, SCH, CHUNK)
  xp = jnp.zeros((NPAD, D_IN), jnp.float32).at[:N].set(x)

  onehot = jnp.zeros((CHUNK, 16), jnp.float32).at[:, 0].set(1.0)
  z16 = jnp.zeros((RPS, 16), jnp.float32)
  z64 = jnp.zeros((RPS, D_HID // 2), jnp.float32)
  z32 = jnp.zeros((RPS, N_CLS // 2), jnp.float32)

  deg = _deg_kernel(dst4, onehot, z16)
  raw1 = _mm1(xp, W1)

  isd = lax.rsqrt(jnp.maximum(deg[0, :, 0:1] + deg[1, :, 0:1], 1.0))
  g1 = raw1 * isd
  agg1 = _agg128(g1, src4, dst4, z64)

  h1 = jnp.maximum(isd * agg1 + b1, 0.0)
  raw2 = _mm2(h1, W2)
  g2 = raw2 * isd
  agg2 = _agg64(g2, src4, dst4, z32)

  return (isd * agg2 + b2)[:N]
